# Initial kernel scaffold; baseline (speedup 1.0000x reference)
import functools

import jax
import jax.numpy as jnp
from jax import lax
from jax.experimental import pallas as pl
from jax.experimental.pallas import tpu as pltpu
from jax.experimental.pallas import tpu_sc as plsc

N = 10000
E = 320000
D = 128
G = 64

NC = 2
NS = 16
NW = NC * NS
CHUNK = 128
EPW = 10112
KCH = EPW // CHUNK
EP = NW * EPW
PAD = EP - E
NACC = 10240
ZB = NACC // NS
DRAIN = N // NS

BN = 2000
NB = N // BN


def _mesh():
    return plsc.VectorSubcoreMesh(core_axis_name="c", subcore_axis_name="s")


def _sc_degree(dstp):
    ones_pay = jnp.ones((CHUNK, 16), jnp.float32)
    zer = jnp.zeros((ZB, 16), jnp.float32)

    @functools.partial(
        pl.kernel,
        out_type=jax.ShapeDtypeStruct((NC * N, 16), jnp.float32),
        mesh=_mesh(),
        scratch_types=[
            pltpu.VMEM((KCH, CHUNK), jnp.int32),
            pltpu.VMEM((CHUNK, 16), jnp.float32),
            pltpu.VMEM_SHARED((NACC, 16), jnp.float32),
        ],
    )
    def k(dst_hbm, ones_hbm, z_hbm, out_hbm, dstv, onesv, acc):
        cid = lax.axis_index("c")
        sid = lax.axis_index("s")
        wid = sid * NC + cid
        pltpu.sync_copy(z_hbm, acc.at[pl.ds(sid * ZB, ZB)])
        pltpu.sync_copy(dst_hbm.at[wid], dstv)
        pltpu.sync_copy(ones_hbm, onesv)
        plsc.subcore_barrier()

        @pl.loop(0, KCH)
        def _(j):
            pltpu.sync_copy(onesv, acc.at[dstv.at[j]], add=True)

        plsc.subcore_barrier()
        pltpu.sync_copy(
            acc.at[pl.ds(sid * DRAIN, DRAIN)],
            out_hbm.at[pl.ds(cid * N + sid * DRAIN, DRAIN)],
        )

    return k(dstp, ones_pay, zer)


def _sc_edge_pass(hs, srcp, dstp):
    zer = jnp.zeros((ZB, D), jnp.float32)

    @functools.partial(
        pl.kernel,
        out_type=jax.ShapeDtypeStruct((NC * N, D), jnp.float32),
        mesh=_mesh(),
        scratch_types=[
            pltpu.VMEM((KCH, CHUNK), jnp.int32),
            pltpu.VMEM((KCH, CHUNK), jnp.int32),
            pltpu.VMEM((CHUNK, D), jnp.float32),
            pltpu.VMEM_SHARED((NACC, D), jnp.float32),
            pltpu.SemaphoreType.DMA,
        ],
    )
    def k(hs_hbm, src_hbm, dst_hbm, z_hbm, out_hbm, srcv, dstv, buf, acc, sem):
        cid = lax.axis_index("c")
        sid = lax.axis_index("s")
        wid = sid * NC + cid
        pltpu.sync_copy(z_hbm, acc.at[pl.ds(sid * ZB, ZB)])
        pltpu.sync_copy(src_hbm.at[wid], srcv)
        pltpu.sync_copy(dst_hbm.at[wid], dstv)
        plsc.subcore_barrier()

        @pl.loop(0, KCH)
        def _(j):
            pltpu.async_copy(hs_hbm.at[srcv.at[j]], buf, sem).wait()
            pltpu.sync_copy(buf, acc.at[dstv.at[j]], add=True)

        plsc.subcore_barrier()
        pltpu.sync_copy(
            acc.at[pl.ds(sid * DRAIN, DRAIN)],
            out_hbm.at[pl.ds(cid * N + sid * DRAIN, DRAIN)],
        )

    return k(hs, srcp, dstp, zer)


def _deg_inv(dp_blk):
    deg = jnp.sum(dp_blk[0] + dp_blk[1], axis=1) * (1.0 / 16.0) + 1.0
    return 1.0 / jnp.sqrt(deg)


def _tc_in(x, W1, dp):
    def body(x_ref, w_ref, p_ref, o_ref):
        dinv = _deg_inv(p_ref[...])
        h = jnp.dot(x_ref[...], w_ref[...], preferred_element_type=jnp.float32)
        o_ref[...] = h * dinv[:, None]

    return pl.pallas_call(
        body,
        grid=(NB,),
        in_specs=[
            pl.BlockSpec((BN, D), lambda i: (i, 0)),
            pl.BlockSpec((D, D), lambda i: (0, 0)),
            pl.BlockSpec((NC, BN, 16), lambda i: (0, i, 0)),
        ],
        out_specs=pl.BlockSpec((BN, D), lambda i: (i, 0)),
        out_shape=jax.ShapeDtypeStruct((N, D), jnp.float32),
    )(x, W1, dp)


def _tc_mid(p, hs1, dp, W2, b1r):
    def body(p_ref, hs_ref, dp_ref, w_ref, b_ref, o_ref):
        dinv = _deg_inv(dp_ref[...])
        pp = p_ref[...]
        agg = pp[0] + pp[1] + hs_ref[...]
        x2 = jnp.maximum(agg * dinv[:, None] + b_ref[...], 0.0)
        h2 = jnp.dot(x2, w_ref[...], preferred_element_type=jnp.float32)
        o_ref[...] = h2 * dinv[:, None]

    return pl.pallas_call(
        body,
        grid=(NB,),
        in_specs=[
            pl.BlockSpec((NC, BN, D), lambda i: (0, i, 0)),
            pl.BlockSpec((BN, D), lambda i: (i, 0)),
            pl.BlockSpec((NC, BN, 16), lambda i: (0, i, 0)),
            pl.BlockSpec((D, D), lambda i: (0, 0)),
            pl.BlockSpec((1, D), lambda i: (0, 0)),
        ],
        out_specs=pl.BlockSpec((BN, D), lambda i: (i, 0)),
        out_shape=jax.ShapeDtypeStruct((N, D), jnp.float32),
    )(p, hs1, dp, W2, b1r)


def _tc_final(q, hs2, dp, b2r, batch3):
    def body(q_ref, hs_ref, dp_ref, b_ref, bt_ref, g_ref, sums, cnts):
        i = pl.program_id(0)
        dinv = _deg_inv(dp_ref[...])
        qq = q_ref[...]
        out = (qq[0] + qq[1] + hs_ref[...]) * dinv[:, None] + b_ref[...]
        seg = bt_ref[0, 0, :]
        onehot = (
            lax.broadcasted_iota(jnp.int32, (G, BN), 0) == seg[None, :]
        ).astype(jnp.float32)
        s = jnp.dot(onehot, out, preferred_element_type=jnp.float32)
        c = jnp.sum(onehot, axis=1)[:, None]

        @pl.when(i == 0)
        def _():
            sums[...] = jnp.zeros_like(sums)
            cnts[...] = jnp.zeros_like(cnts)

        sums[...] += s
        cnts[...] += jnp.broadcast_to(c, (G, D))

        @pl.when(i == NB - 1)
        def _():
            g_ref[...] = sums[...] / jnp.maximum(cnts[...], 1.0)

    return pl.pallas_call(
        body,
        grid=(NB,),
        in_specs=[
            pl.BlockSpec((NC, BN, D), lambda i: (0, i, 0)),
            pl.BlockSpec((BN, D), lambda i: (i, 0)),
            pl.BlockSpec((NC, BN, 16), lambda i: (0, i, 0)),
            pl.BlockSpec((1, D), lambda i: (0, 0)),
            pl.BlockSpec((1, 1, BN), lambda i: (i, 0, 0)),
        ],
        out_specs=pl.BlockSpec((G, D), lambda i: (0, 0)),
        out_shape=jax.ShapeDtypeStruct((G, D), jnp.float32),
        scratch_shapes=[
            pltpu.VMEM((G, D), jnp.float32),
            pltpu.VMEM((G, D), jnp.float32),
        ],
    )(q, hs2, dp, b2r, batch3)


def kernel(x, edge_index, batch, W1, b1, W2, b2):
    src = edge_index[0]
    dst = edge_index[1]
    ar = jnp.arange(PAD, dtype=jnp.int32)
    srcp = jnp.concatenate([src, ar % N]).reshape(NW, KCH, CHUNK)
    dstp = jnp.concatenate([dst, N + ar % (NACC - N)]).reshape(NW, KCH, CHUNK)

    dp = _sc_degree(dstp).reshape(NC, N, 16)
    hs1 = _tc_in(x, W1, dp)
    p1 = _sc_edge_pass(hs1, srcp, dstp).reshape(NC, N, D)
    hs2 = _tc_mid(p1, hs1, dp, W2, b1.reshape(1, D))
    p2 = _sc_edge_pass(hs2, srcp, dstp).reshape(NC, N, D)
    return _tc_final(p2, hs2, dp, b2.reshape(1, D), batch.reshape(NB, 1, BN))


# speedup vs baseline: 20.5264x; 20.5264x over previous
import functools

import jax
import jax.numpy as jnp
from jax import lax
from jax.experimental import pallas as pl
from jax.experimental.pallas import tpu as pltpu
from jax.experimental.pallas import tpu_sc as plsc

N = 10000
E = 320000
D = 128
G = 64

NC = 2
NS = 16
NW = NC * NS
CHUNK = 128
EPW = 10112
KCH = EPW // CHUNK
EP = NW * EPW
PAD = EP - E
NACC = 10240
ZB = NACC // NS
DRAIN = NACC // NS

BN = 2000
NB = N // BN


def _mesh():
    return plsc.VectorSubcoreMesh(core_axis_name="c", subcore_axis_name="s")


def _sc_degree(dstp):
    ones_pay = jnp.ones((CHUNK, D), jnp.float32)
    zer = jnp.zeros((ZB, D), jnp.float32)

    @functools.partial(
        pl.kernel,
        out_type=jax.ShapeDtypeStruct((NC * NACC, D), jnp.float32),
        mesh=_mesh(),
        scratch_types=[
            pltpu.VMEM((KCH, CHUNK), jnp.int32),
            pltpu.VMEM((CHUNK, D), jnp.float32),
            pltpu.VMEM_SHARED((NACC, D), jnp.float32),
        ],
    )
    def k(dst_hbm, ones_hbm, z_hbm, out_hbm, dstv, onesv, acc):
        cid = lax.axis_index("c")
        sid = lax.axis_index("s")
        wid = sid * NC + cid
        pltpu.sync_copy(z_hbm, acc.at[pl.ds(sid * ZB, ZB)])
        pltpu.sync_copy(dst_hbm.at[wid], dstv)
        pltpu.sync_copy(ones_hbm, onesv)
        plsc.subcore_barrier()

        @pl.loop(0, KCH)
        def _(j):
            pltpu.sync_copy(onesv, acc.at[dstv.at[j]], add=True)

        plsc.subcore_barrier()
        pltpu.sync_copy(
            acc.at[pl.ds(sid * DRAIN, DRAIN)],
            out_hbm.at[pl.ds(cid * NACC + sid * DRAIN, DRAIN)],
        )

    return k(dstp, ones_pay, zer)


def _sc_edge_pass(hs, srcp, dstp):
    zer = jnp.zeros((ZB, D), jnp.float32)

    @functools.partial(
        pl.kernel,
        out_type=jax.ShapeDtypeStruct((NC * NACC, D), jnp.float32),
        mesh=_mesh(),
        scratch_types=[
            pltpu.VMEM((KCH, CHUNK), jnp.int32),
            pltpu.VMEM((KCH, CHUNK), jnp.int32),
            pltpu.VMEM((CHUNK, D), jnp.float32),
            pltpu.VMEM_SHARED((NACC, D), jnp.float32),
            pltpu.SemaphoreType.DMA,
        ],
    )
    def k(hs_hbm, src_hbm, dst_hbm, z_hbm, out_hbm, srcv, dstv, buf, acc, sem):
        cid = lax.axis_index("c")
        sid = lax.axis_index("s")
        wid = sid * NC + cid
        pltpu.sync_copy(z_hbm, acc.at[pl.ds(sid * ZB, ZB)])
        pltpu.sync_copy(src_hbm.at[wid], srcv)
        pltpu.sync_copy(dst_hbm.at[wid], dstv)
        plsc.subcore_barrier()

        @pl.loop(0, KCH)
        def _(j):
            pltpu.async_copy(hs_hbm.at[srcv.at[j]], buf, sem).wait()
            pltpu.sync_copy(buf, acc.at[dstv.at[j]], add=True)

        plsc.subcore_barrier()
        pltpu.sync_copy(
            acc.at[pl.ds(sid * DRAIN, DRAIN)],
            out_hbm.at[pl.ds(cid * NACC + sid * DRAIN, DRAIN)],
        )

    return k(hs, srcp, dstp, zer)


def _deg_inv(dp_blk):
    deg = jnp.sum(dp_blk[0] + dp_blk[1], axis=1) * (1.0 / D) + 1.0
    return 1.0 / jnp.sqrt(deg)


def _tc_in(x, W1, dp):
    def body(x_ref, w_ref, p_ref, o_ref):
        dinv = _deg_inv(p_ref[...])
        h = jnp.dot(x_ref[...], w_ref[...], preferred_element_type=jnp.float32)
        o_ref[...] = h * dinv[:, None]

    return pl.pallas_call(
        body,
        grid=(NB,),
        in_specs=[
            pl.BlockSpec((BN, D), lambda i: (i, 0)),
            pl.BlockSpec((D, D), lambda i: (0, 0)),
            pl.BlockSpec((NC, BN, D), lambda i: (0, i, 0)),
        ],
        out_specs=pl.BlockSpec((BN, D), lambda i: (i, 0)),
        out_shape=jax.ShapeDtypeStruct((N, D), jnp.float32),
    )(x, W1, dp)


def _tc_mid(p, hs1, dp, W2, b1r):
    def body(p_ref, hs_ref, dp_ref, w_ref, b_ref, o_ref):
        dinv = _deg_inv(dp_ref[...])
        pp = p_ref[...]
        agg = pp[0] + pp[1] + hs_ref[...]
        x2 = jnp.maximum(agg * dinv[:, None] + b_ref[...], 0.0)
        h2 = jnp.dot(x2, w_ref[...], preferred_element_type=jnp.float32)
        o_ref[...] = h2 * dinv[:, None]

    return pl.pallas_call(
        body,
        grid=(NB,),
        in_specs=[
            pl.BlockSpec((NC, BN, D), lambda i: (0, i, 0)),
            pl.BlockSpec((BN, D), lambda i: (i, 0)),
            pl.BlockSpec((NC, BN, D), lambda i: (0, i, 0)),
            pl.BlockSpec((D, D), lambda i: (0, 0)),
            pl.BlockSpec((1, D), lambda i: (0, 0)),
        ],
        out_specs=pl.BlockSpec((BN, D), lambda i: (i, 0)),
        out_shape=jax.ShapeDtypeStruct((N, D), jnp.float32),
    )(p, hs1, dp, W2, b1r)


def _tc_final(q, hs2, dp, b2r, batch3):
    def body(q_ref, hs_ref, dp_ref, b_ref, bt_ref, g_ref, sums, cnts):
        i = pl.program_id(0)
        dinv = _deg_inv(dp_ref[...])
        qq = q_ref[...]
        out = (qq[0] + qq[1] + hs_ref[...]) * dinv[:, None] + b_ref[...]
        seg = bt_ref[0, 0, :]
        onehot = (
            lax.broadcasted_iota(jnp.int32, (G, BN), 0) == seg[None, :]
        ).astype(jnp.float32)
        s = jnp.dot(onehot, out, preferred_element_type=jnp.float32)
        c = jnp.sum(onehot, axis=1)[:, None]

        @pl.when(i == 0)
        def _():
            sums[...] = jnp.zeros_like(sums)
            cnts[...] = jnp.zeros_like(cnts)

        sums[...] += s
        cnts[...] += jnp.broadcast_to(c, (G, D))

        @pl.when(i == NB - 1)
        def _():
            g_ref[...] = sums[...] / jnp.maximum(cnts[...], 1.0)

    return pl.pallas_call(
        body,
        grid=(NB,),
        in_specs=[
            pl.BlockSpec((NC, BN, D), lambda i: (0, i, 0)),
            pl.BlockSpec((BN, D), lambda i: (i, 0)),
            pl.BlockSpec((NC, BN, D), lambda i: (0, i, 0)),
            pl.BlockSpec((1, D), lambda i: (0, 0)),
            pl.BlockSpec((1, 1, BN), lambda i: (i, 0, 0)),
        ],
        out_specs=pl.BlockSpec((G, D), lambda i: (0, 0)),
        out_shape=jax.ShapeDtypeStruct((G, D), jnp.float32),
        scratch_shapes=[
            pltpu.VMEM((G, D), jnp.float32),
            pltpu.VMEM((G, D), jnp.float32),
        ],
    )(q, hs2, dp, b2r, batch3)


def kernel(x, edge_index, batch, W1, b1, W2, b2):
    src = edge_index[0]
    dst = edge_index[1]
    ar = jnp.arange(PAD, dtype=jnp.int32)
    srcp = jnp.concatenate([src, ar % N]).reshape(NW, KCH, CHUNK)
    dstp = jnp.concatenate([dst, N + ar % (NACC - N)]).reshape(NW, KCH, CHUNK)

    dp = _sc_degree(dstp).reshape(NC, NACC, D)
    hs1 = _tc_in(x, W1, dp)
    p1 = _sc_edge_pass(hs1, srcp, dstp).reshape(NC, NACC, D)
    hs2 = _tc_mid(p1, hs1, dp, W2, b1.reshape(1, D))
    p2 = _sc_edge_pass(hs2, srcp, dstp).reshape(NC, NACC, D)
    return _tc_final(p2, hs2, dp, b2.reshape(1, D), batch.reshape(NB, 1, BN))


# speedup vs baseline: 27.4520x; 1.3374x over previous
import functools

import jax
import jax.numpy as jnp
from jax import lax
from jax.experimental import pallas as pl
from jax.experimental.pallas import tpu as pltpu
from jax.experimental.pallas import tpu_sc as plsc

N = 10000
E = 320000
D = 128
G = 64

NC = 2
NS = 16
NW = NC * NS
CHUNK = 128
EPW = 10240
KCH = EPW // CHUNK
EP = NW * EPW
PAD = EP - E
NACC = 10240
ZB = NACC // NS
DRAIN = NACC // NS

NSEG = 2
KS = KCH // NSEG

BN = 2000
NB = N // BN


def _mesh():
    return plsc.VectorSubcoreMesh(core_axis_name="c", subcore_axis_name="s")


def _sc_degree(dstp):
    ones_pay = jnp.ones((CHUNK, D), jnp.float32)
    zer = jnp.zeros((ZB, D), jnp.float32)

    @functools.partial(
        pl.kernel,
        out_type=jax.ShapeDtypeStruct((NC * NACC, D), jnp.float32),
        mesh=_mesh(),
        scratch_types=[
            pltpu.VMEM((KCH, CHUNK), jnp.int32),
            pltpu.VMEM((CHUNK, D), jnp.float32),
            pltpu.VMEM_SHARED((NACC, D), jnp.float32),
        ],
    )
    def k(dst_hbm, ones_hbm, z_hbm, out_hbm, dstv, onesv, acc):
        cid = lax.axis_index("c")
        sid = lax.axis_index("s")
        wid = sid * NC + cid
        pltpu.sync_copy(z_hbm, acc.at[pl.ds(sid * ZB, ZB)])
        pltpu.sync_copy(dst_hbm.at[wid], dstv)
        pltpu.sync_copy(ones_hbm, onesv)
        plsc.subcore_barrier()

        @pl.loop(0, KCH)
        def _(j):
            pltpu.sync_copy(onesv, acc.at[dstv.at[j]], add=True)

        plsc.subcore_barrier()
        pltpu.sync_copy(
            acc.at[pl.ds(sid * DRAIN, DRAIN)],
            out_hbm.at[pl.ds(cid * NACC + sid * DRAIN, DRAIN)],
        )

    return k(dstp, ones_pay, zer)


def _sc_edge_pass(hs, srcp, dstp):
    zer = jnp.zeros((ZB, D), jnp.float32)

    @functools.partial(
        pl.kernel,
        out_type=jax.ShapeDtypeStruct((NC * NACC, D), jnp.float32),
        mesh=_mesh(),
        scratch_types=[
            pltpu.VMEM((KS, CHUNK), jnp.int32),
            pltpu.VMEM((KS, CHUNK), jnp.int32),
            pltpu.VMEM((CHUNK, D), jnp.float32),
            pltpu.VMEM((CHUNK, D), jnp.float32),
            pltpu.VMEM_SHARED((NACC, D), jnp.float32),
            pltpu.SemaphoreType.DMA,
            pltpu.SemaphoreType.DMA,
        ],
    )
    def k(hs_hbm, src_hbm, dst_hbm, z_hbm, out_hbm,
          srcv, dstv, buf0, buf1, acc, sem0, sem1):
        cid = lax.axis_index("c")
        sid = lax.axis_index("s")
        wid = sid * NC + cid
        pltpu.sync_copy(z_hbm, acc.at[pl.ds(sid * ZB, ZB)])
        plsc.subcore_barrier()

        @pl.loop(0, NSEG)
        def _(sg):
            pltpu.sync_copy(src_hbm.at[wid].at[pl.ds(sg * KS, KS)], srcv)
            pltpu.sync_copy(dst_hbm.at[wid].at[pl.ds(sg * KS, KS)], dstv)
            pltpu.async_copy(hs_hbm.at[srcv.at[0]], buf0, sem0)

            @pl.loop(0, KS // 2)
            def _(t):
                j0 = t * 2
                j1 = j0 + 1
                pltpu.async_copy(hs_hbm.at[srcv.at[j1]], buf1, sem1)
                pltpu.make_async_copy(hs_hbm.at[srcv.at[j0]], buf0, sem0).wait()
                pltpu.sync_copy(buf0, acc.at[dstv.at[j0]], add=True)

                @pl.when(j0 + 2 < KS)
                def _():
                    pltpu.async_copy(hs_hbm.at[srcv.at[j0 + 2]], buf0, sem0)

                pltpu.make_async_copy(hs_hbm.at[srcv.at[j1]], buf1, sem1).wait()
                pltpu.sync_copy(buf1, acc.at[dstv.at[j1]], add=True)

        plsc.subcore_barrier()
        pltpu.sync_copy(
            acc.at[pl.ds(sid * DRAIN, DRAIN)],
            out_hbm.at[pl.ds(cid * NACC + sid * DRAIN, DRAIN)],
        )

    return k(hs, srcp, dstp, zer)


def _deg_inv(dp_blk):
    deg = jnp.sum(dp_blk[0] + dp_blk[1], axis=1) * (1.0 / D) + 1.0
    return 1.0 / jnp.sqrt(deg)


def _tc_in(x, W1, dp):
    def body(x_ref, w_ref, p_ref, o_ref):
        dinv = _deg_inv(p_ref[...])
        h = jnp.dot(x_ref[...], w_ref[...], preferred_element_type=jnp.float32)
        o_ref[...] = h * dinv[:, None]

    return pl.pallas_call(
        body,
        grid=(NB,),
        in_specs=[
            pl.BlockSpec((BN, D), lambda i: (i, 0)),
            pl.BlockSpec((D, D), lambda i: (0, 0)),
            pl.BlockSpec((NC, BN, D), lambda i: (0, i, 0)),
        ],
        out_specs=pl.BlockSpec((BN, D), lambda i: (i, 0)),
        out_shape=jax.ShapeDtypeStruct((N, D), jnp.float32),
    )(x, W1, dp)


def _tc_mid(p, hs1, dp, W2, b1r):
    def body(p_ref, hs_ref, dp_ref, w_ref, b_ref, o_ref):
        dinv = _deg_inv(dp_ref[...])
        pp = p_ref[...]
        agg = pp[0] + pp[1] + hs_ref[...]
        x2 = jnp.maximum(agg * dinv[:, None] + b_ref[...], 0.0)
        h2 = jnp.dot(x2, w_ref[...], preferred_element_type=jnp.float32)
        o_ref[...] = h2 * dinv[:, None]

    return pl.pallas_call(
        body,
        grid=(NB,),
        in_specs=[
            pl.BlockSpec((NC, BN, D), lambda i: (0, i, 0)),
            pl.BlockSpec((BN, D), lambda i: (i, 0)),
            pl.BlockSpec((NC, BN, D), lambda i: (0, i, 0)),
            pl.BlockSpec((D, D), lambda i: (0, 0)),
            pl.BlockSpec((1, D), lambda i: (0, 0)),
        ],
        out_specs=pl.BlockSpec((BN, D), lambda i: (i, 0)),
        out_shape=jax.ShapeDtypeStruct((N, D), jnp.float32),
    )(p, hs1, dp, W2, b1r)


def _tc_final(q, hs2, dp, b2r, batch3):
    def body(q_ref, hs_ref, dp_ref, b_ref, bt_ref, g_ref, sums, cnts):
        i = pl.program_id(0)
        dinv = _deg_inv(dp_ref[...])
        qq = q_ref[...]
        out = (qq[0] + qq[1] + hs_ref[...]) * dinv[:, None] + b_ref[...]
        seg = bt_ref[0, 0, :]
        onehot = (
            lax.broadcasted_iota(jnp.int32, (G, BN), 0) == seg[None, :]
        ).astype(jnp.float32)
        s = jnp.dot(onehot, out, preferred_element_type=jnp.float32)
        c = jnp.sum(onehot, axis=1)[:, None]

        @pl.when(i == 0)
        def _():
            sums[...] = jnp.zeros_like(sums)
            cnts[...] = jnp.zeros_like(cnts)

        sums[...] += s
        cnts[...] += jnp.broadcast_to(c, (G, D))

        @pl.when(i == NB - 1)
        def _():
            g_ref[...] = sums[...] / jnp.maximum(cnts[...], 1.0)

    return pl.pallas_call(
        body,
        grid=(NB,),
        in_specs=[
            pl.BlockSpec((NC, BN, D), lambda i: (0, i, 0)),
            pl.BlockSpec((BN, D), lambda i: (i, 0)),
            pl.BlockSpec((NC, BN, D), lambda i: (0, i, 0)),
            pl.BlockSpec((1, D), lambda i: (0, 0)),
            pl.BlockSpec((1, 1, BN), lambda i: (i, 0, 0)),
        ],
        out_specs=pl.BlockSpec((G, D), lambda i: (0, 0)),
        out_shape=jax.ShapeDtypeStruct((G, D), jnp.float32),
        scratch_shapes=[
            pltpu.VMEM((G, D), jnp.float32),
            pltpu.VMEM((G, D), jnp.float32),
        ],
    )(q, hs2, dp, b2r, batch3)


def kernel(x, edge_index, batch, W1, b1, W2, b2):
    src = edge_index[0]
    dst = edge_index[1]
    ar = jnp.arange(PAD, dtype=jnp.int32)
    srcp = jnp.concatenate([src, ar % N]).reshape(NW, KCH, CHUNK)
    dstp = jnp.concatenate([dst, N + ar % (NACC - N)]).reshape(NW, KCH, CHUNK)

    dp = _sc_degree(dstp).reshape(NC, NACC, D)
    hs1 = _tc_in(x, W1, dp)
    p1 = _sc_edge_pass(hs1, srcp, dstp).reshape(NC, NACC, D)
    hs2 = _tc_mid(p1, hs1, dp, W2, b1.reshape(1, D))
    p2 = _sc_edge_pass(hs2, srcp, dstp).reshape(NC, NACC, D)
    return _tc_final(p2, hs2, dp, b2.reshape(1, D), batch.reshape(NB, 1, BN))


# speedup vs baseline: 27.7096x; 1.0094x over previous
import functools

import jax
import jax.numpy as jnp
from jax import lax
from jax.experimental import pallas as pl
from jax.experimental.pallas import tpu as pltpu
from jax.experimental.pallas import tpu_sc as plsc

N = 10000
E = 320000
D = 128
G = 64

NC = 2
NS = 16
NW = NC * NS
CHUNK = 128
EPW = 10240
KCH = EPW // CHUNK
EP = NW * EPW
PAD = EP - E
NACC = 10240
ZB = NACC // NS
DRAIN = NACC // NS

NSEG = 2
KS = KCH // NSEG

BN = 2000
NB = N // BN


def _mesh():
    return plsc.VectorSubcoreMesh(core_axis_name="c", subcore_axis_name="s")


def _sc_degree(dstp):
    ones_pay = jnp.ones((NW * CHUNK, D), jnp.float32)
    zer = jnp.zeros((NC * NACC, D), jnp.float32)

    @functools.partial(
        pl.kernel,
        out_type=jax.ShapeDtypeStruct((NC * NACC, D), jnp.float32),
        mesh=_mesh(),
        scratch_types=[
            pltpu.VMEM((KCH, CHUNK), jnp.int32),
            pltpu.VMEM((CHUNK, D), jnp.float32),
            pltpu.VMEM_SHARED((NACC, D), jnp.float32),
        ],
    )
    def k(dst_hbm, ones_hbm, z_hbm, out_hbm, dstv, onesv, acc):
        cid = lax.axis_index("c")
        sid = lax.axis_index("s")
        wid = sid * NC + cid
        pltpu.sync_copy(z_hbm.at[pl.ds(cid * NACC + sid * ZB, ZB)],
                        acc.at[pl.ds(sid * ZB, ZB)])
        pltpu.sync_copy(dst_hbm.at[wid], dstv)
        pltpu.sync_copy(ones_hbm.at[pl.ds(wid * CHUNK, CHUNK)], onesv)
        plsc.subcore_barrier()

        @pl.loop(0, KCH)
        def _(j):
            pltpu.sync_copy(onesv, acc.at[dstv.at[j]], add=True)

        plsc.subcore_barrier()
        pltpu.sync_copy(
            acc.at[pl.ds(sid * DRAIN, DRAIN)],
            out_hbm.at[pl.ds(cid * NACC + sid * DRAIN, DRAIN)],
        )

    return k(dstp, ones_pay, zer)


def _sc_edge_pass(hs, srcp, dstp):
    zer = jnp.zeros((NC * NACC, D), jnp.float32)

    @functools.partial(
        pl.kernel,
        out_type=jax.ShapeDtypeStruct((NC * NACC, D), jnp.float32),
        mesh=_mesh(),
        scratch_types=[
            pltpu.VMEM((KS, CHUNK), jnp.int32),
            pltpu.VMEM((KS, CHUNK), jnp.int32),
            pltpu.VMEM((CHUNK, D), jnp.float32),
            pltpu.VMEM((CHUNK, D), jnp.float32),
            pltpu.VMEM_SHARED((NACC, D), jnp.float32),
            pltpu.SemaphoreType.DMA,
            pltpu.SemaphoreType.DMA,
        ],
    )
    def k(hs_hbm, src_hbm, dst_hbm, z_hbm, out_hbm,
          srcv, dstv, buf0, buf1, acc, sem0, sem1):
        cid = lax.axis_index("c")
        sid = lax.axis_index("s")
        wid = sid * NC + cid
        pltpu.sync_copy(z_hbm.at[pl.ds(cid * NACC + sid * ZB, ZB)],
                        acc.at[pl.ds(sid * ZB, ZB)])
        plsc.subcore_barrier()

        @pl.loop(0, NSEG)
        def _(sg):
            pltpu.sync_copy(src_hbm.at[wid].at[pl.ds(sg * KS, KS)], srcv)
            pltpu.sync_copy(dst_hbm.at[wid].at[pl.ds(sg * KS, KS)], dstv)
            pltpu.async_copy(hs_hbm.at[srcv.at[0]], buf0, sem0)

            @pl.loop(0, KS // 2)
            def _(t):
                j0 = t * 2
                j1 = j0 + 1
                pltpu.async_copy(hs_hbm.at[srcv.at[j1]], buf1, sem1)
                pltpu.make_async_copy(hs_hbm.at[srcv.at[j0]], buf0, sem0).wait()
                pltpu.sync_copy(buf0, acc.at[dstv.at[j0]], add=True)

                @pl.when(j0 + 2 < KS)
                def _():
                    pltpu.async_copy(hs_hbm.at[srcv.at[j0 + 2]], buf0, sem0)

                pltpu.make_async_copy(hs_hbm.at[srcv.at[j1]], buf1, sem1).wait()
                pltpu.sync_copy(buf1, acc.at[dstv.at[j1]], add=True)

        plsc.subcore_barrier()
        pltpu.sync_copy(
            acc.at[pl.ds(sid * DRAIN, DRAIN)],
            out_hbm.at[pl.ds(cid * NACC + sid * DRAIN, DRAIN)],
        )

    return k(hs, srcp, dstp, zer)


def _deg_inv(dp_blk):
    deg = jnp.sum(dp_blk[0] + dp_blk[1], axis=1) * (1.0 / D) + 1.0
    return 1.0 / jnp.sqrt(deg)


def _tc_in(x, W1, dp):
    def body(x_ref, w_ref, p_ref, o_ref):
        dinv = _deg_inv(p_ref[...])
        h = jnp.dot(x_ref[...], w_ref[...], preferred_element_type=jnp.float32)
        o_ref[...] = h * dinv[:, None]

    return pl.pallas_call(
        body,
        grid=(NB,),
        in_specs=[
            pl.BlockSpec((BN, D), lambda i: (i, 0)),
            pl.BlockSpec((D, D), lambda i: (0, 0)),
            pl.BlockSpec((NC, BN, D), lambda i: (0, i, 0)),
        ],
        out_specs=pl.BlockSpec((BN, D), lambda i: (i, 0)),
        out_shape=jax.ShapeDtypeStruct((N, D), jnp.float32),
    )(x, W1, dp)


def _tc_mid(p, hs1, dp, W2, b1r):
    def body(p_ref, hs_ref, dp_ref, w_ref, b_ref, o_ref):
        dinv = _deg_inv(dp_ref[...])
        pp = p_ref[...]
        agg = pp[0] + pp[1] + hs_ref[...]
        x2 = jnp.maximum(agg * dinv[:, None] + b_ref[...], 0.0)
        h2 = jnp.dot(x2, w_ref[...], preferred_element_type=jnp.float32)
        o_ref[...] = h2 * dinv[:, None]

    return pl.pallas_call(
        body,
        grid=(NB,),
        in_specs=[
            pl.BlockSpec((NC, BN, D), lambda i: (0, i, 0)),
            pl.BlockSpec((BN, D), lambda i: (i, 0)),
            pl.BlockSpec((NC, BN, D), lambda i: (0, i, 0)),
            pl.BlockSpec((D, D), lambda i: (0, 0)),
            pl.BlockSpec((1, D), lambda i: (0, 0)),
        ],
        out_specs=pl.BlockSpec((BN, D), lambda i: (i, 0)),
        out_shape=jax.ShapeDtypeStruct((N, D), jnp.float32),
    )(p, hs1, dp, W2, b1r)


def _tc_final(q, hs2, dp, b2r, batch3):
    def body(q_ref, hs_ref, dp_ref, b_ref, bt_ref, g_ref, sums, cnts):
        i = pl.program_id(0)
        dinv = _deg_inv(dp_ref[...])
        qq = q_ref[...]
        out = (qq[0] + qq[1] + hs_ref[...]) * dinv[:, None] + b_ref[...]
        seg = bt_ref[0, 0, :]
        onehot = (
            lax.broadcasted_iota(jnp.int32, (G, BN), 0) == seg[None, :]
        ).astype(jnp.float32)
        s = jnp.dot(onehot, out, preferred_element_type=jnp.float32)
        c = jnp.sum(onehot, axis=1)[:, None]

        @pl.when(i == 0)
        def _():
            sums[...] = jnp.zeros_like(sums)
            cnts[...] = jnp.zeros_like(cnts)

        sums[...] += s
        cnts[...] += jnp.broadcast_to(c, (G, D))

        @pl.when(i == NB - 1)
        def _():
            g_ref[...] = sums[...] / jnp.maximum(cnts[...], 1.0)

    return pl.pallas_call(
        body,
        grid=(NB,),
        in_specs=[
            pl.BlockSpec((NC, BN, D), lambda i: (0, i, 0)),
            pl.BlockSpec((BN, D), lambda i: (i, 0)),
            pl.BlockSpec((NC, BN, D), lambda i: (0, i, 0)),
            pl.BlockSpec((1, D), lambda i: (0, 0)),
            pl.BlockSpec((1, 1, BN), lambda i: (i, 0, 0)),
        ],
        out_specs=pl.BlockSpec((G, D), lambda i: (0, 0)),
        out_shape=jax.ShapeDtypeStruct((G, D), jnp.float32),
        scratch_shapes=[
            pltpu.VMEM((G, D), jnp.float32),
            pltpu.VMEM((G, D), jnp.float32),
        ],
    )(q, hs2, dp, b2r, batch3)


def kernel(x, edge_index, batch, W1, b1, W2, b2):
    src = edge_index[0]
    dst = edge_index[1]
    ar = jnp.arange(PAD, dtype=jnp.int32)
    srcp = jnp.concatenate([src, ar % N]).reshape(NW, KCH, CHUNK)
    dstp = jnp.concatenate([dst, N + ar % (NACC - N)]).reshape(NW, KCH, CHUNK)

    dp = _sc_degree(dstp).reshape(NC, NACC, D)
    hs1 = _tc_in(x, W1, dp)
    p1 = _sc_edge_pass(hs1, srcp, dstp).reshape(NC, NACC, D)
    hs2 = _tc_mid(p1, hs1, dp, W2, b1.reshape(1, D))
    p2 = _sc_edge_pass(hs2, srcp, dstp).reshape(NC, NACC, D)
    return _tc_final(p2, hs2, dp, b2.reshape(1, D), batch.reshape(NB, 1, BN))


# speedup vs baseline: 27.8977x; 1.0068x over previous
import functools

import jax
import jax.numpy as jnp
from jax import lax
from jax.experimental import pallas as pl
from jax.experimental.pallas import tpu as pltpu
from jax.experimental.pallas import tpu_sc as plsc

N = 10000
E = 320000
D = 128
G = 64

NC = 2
NS = 16
NW = NC * NS
CHUNK = 128
EPW = 10240
KCH = EPW // CHUNK
EP = NW * EPW
PAD = EP - E
NACC = 10240
ZB = NACC // NS
DRAIN = NACC // NS

NSEG = 2
KS = KCH // NSEG

BN = 2000
NB = N // BN


def _mesh():
    return plsc.VectorSubcoreMesh(core_axis_name="c", subcore_axis_name="s")


def _sc_degree(dstp):
    ones_pay = jnp.ones((NW * CHUNK, D), jnp.float32)
    zer = jnp.zeros((NC * NACC, D), jnp.float32)

    @functools.partial(
        pl.kernel,
        out_type=jax.ShapeDtypeStruct((NC * NACC, D), jnp.float32),
        mesh=_mesh(),
        scratch_types=[
            pltpu.VMEM((KCH, CHUNK), jnp.int32),
            pltpu.VMEM((CHUNK, D), jnp.float32),
            pltpu.VMEM_SHARED((NACC, D), jnp.float32),
        ],
    )
    def k(dst_hbm, ones_hbm, z_hbm, out_hbm, dstv, onesv, acc):
        cid = lax.axis_index("c")
        sid = lax.axis_index("s")
        wid = sid * NC + cid
        pltpu.sync_copy(z_hbm.at[pl.ds(cid * NACC + sid * ZB, ZB)],
                        acc.at[pl.ds(sid * ZB, ZB)])
        pltpu.sync_copy(dst_hbm.at[wid], dstv)
        pltpu.sync_copy(ones_hbm.at[pl.ds(wid * CHUNK, CHUNK)], onesv)
        plsc.subcore_barrier()

        @pl.loop(0, KCH)
        def _(j):
            pltpu.sync_copy(onesv, acc.at[dstv.at[j]], add=True)

        plsc.subcore_barrier()
        pltpu.sync_copy(
            acc.at[pl.ds(sid * DRAIN, DRAIN)],
            out_hbm.at[pl.ds(cid * NACC + sid * DRAIN, DRAIN)],
        )

    return k(dstp, ones_pay, zer)


def _sc_edge_pass(hs, srcp, dstp):
    zer = jnp.zeros((NC * NACC, D), jnp.float32)

    @functools.partial(
        pl.kernel,
        out_type=jax.ShapeDtypeStruct((NC * NACC, D), jnp.float32),
        mesh=_mesh(),
        scratch_types=[
            pltpu.VMEM((KS, CHUNK), jnp.int32),
            pltpu.VMEM((KS, CHUNK), jnp.int32),
            pltpu.VMEM((CHUNK, D), jnp.float32),
            pltpu.VMEM((CHUNK, D), jnp.float32),
            pltpu.VMEM_SHARED((NACC, D), jnp.float32),
            pltpu.SemaphoreType.DMA,
            pltpu.SemaphoreType.DMA,
        ],
    )
    def k(hs_hbm, src_hbm, dst_hbm, z_hbm, out_hbm,
          srcv, dstv, buf0, buf1, acc, sem0, sem1):
        cid = lax.axis_index("c")
        sid = lax.axis_index("s")
        wid = sid * NC + cid
        pltpu.sync_copy(z_hbm.at[pl.ds(cid * NACC + sid * ZB, ZB)],
                        acc.at[pl.ds(sid * ZB, ZB)])
        plsc.subcore_barrier()

        @pl.loop(0, NSEG)
        def _(sg):
            pltpu.sync_copy(src_hbm.at[wid].at[pl.ds(sg * KS, KS)], srcv)
            pltpu.sync_copy(dst_hbm.at[wid].at[pl.ds(sg * KS, KS)], dstv)
            pltpu.async_copy(hs_hbm.at[srcv.at[0]], buf0, sem0)

            @pl.loop(0, KS // 2)
            def _(t):
                j0 = t * 2
                j1 = j0 + 1
                pltpu.async_copy(hs_hbm.at[srcv.at[j1]], buf1, sem1)
                pltpu.make_async_copy(hs_hbm.at[srcv.at[j0]], buf0, sem0).wait()
                pltpu.sync_copy(buf0, acc.at[dstv.at[j0]], add=True)

                @pl.when(j0 + 2 < KS)
                def _():
                    pltpu.async_copy(hs_hbm.at[srcv.at[j0 + 2]], buf0, sem0)

                pltpu.make_async_copy(hs_hbm.at[srcv.at[j1]], buf1, sem1).wait()
                pltpu.sync_copy(buf1, acc.at[dstv.at[j1]], add=True)

        plsc.subcore_barrier()
        pltpu.sync_copy(
            acc.at[pl.ds(sid * DRAIN, DRAIN)],
            out_hbm.at[pl.ds(cid * NACC + sid * DRAIN, DRAIN)],
        )

    return k(hs, srcp, dstp, zer)


def _deg_inv(dp_blk):
    deg = jnp.sum(dp_blk[0] + dp_blk[1], axis=1) * (1.0 / D) + 1.0
    return 1.0 / jnp.sqrt(deg)


def _tc_in(x, W1, dp):
    def body(x_ref, w_ref, p_ref, o_ref, d_ref):
        dinv = _deg_inv(p_ref[...])
        h = jnp.dot(x_ref[...], w_ref[...], preferred_element_type=jnp.float32)
        o_ref[...] = h * dinv[:, None]
        d_ref[0, 0, :] = dinv

    return pl.pallas_call(
        body,
        grid=(NB,),
        in_specs=[
            pl.BlockSpec((BN, D), lambda i: (i, 0)),
            pl.BlockSpec((D, D), lambda i: (0, 0)),
            pl.BlockSpec((NC, BN, D), lambda i: (0, i, 0)),
        ],
        out_specs=[
            pl.BlockSpec((BN, D), lambda i: (i, 0)),
            pl.BlockSpec((1, 1, BN), lambda i: (i, 0, 0)),
        ],
        out_shape=[
            jax.ShapeDtypeStruct((N, D), jnp.float32),
            jax.ShapeDtypeStruct((NB, 1, BN), jnp.float32),
        ],
    )(x, W1, dp)


def _tc_mid(p, hs1, dinv3, W2, b1r):
    def body(p_ref, hs_ref, dp_ref, w_ref, b_ref, o_ref):
        dinv = dp_ref[0, 0, :]
        pp = p_ref[...]
        agg = pp[0] + pp[1] + hs_ref[...]
        x2 = jnp.maximum(agg * dinv[:, None] + b_ref[...], 0.0)
        h2 = jnp.dot(x2, w_ref[...], preferred_element_type=jnp.float32)
        o_ref[...] = h2 * dinv[:, None]

    return pl.pallas_call(
        body,
        grid=(NB,),
        in_specs=[
            pl.BlockSpec((NC, BN, D), lambda i: (0, i, 0)),
            pl.BlockSpec((BN, D), lambda i: (i, 0)),
            pl.BlockSpec((1, 1, BN), lambda i: (i, 0, 0)),
            pl.BlockSpec((D, D), lambda i: (0, 0)),
            pl.BlockSpec((1, D), lambda i: (0, 0)),
        ],
        out_specs=pl.BlockSpec((BN, D), lambda i: (i, 0)),
        out_shape=jax.ShapeDtypeStruct((N, D), jnp.float32),
    )(p, hs1, dinv3, W2, b1r)


def _tc_final(q, hs2, dinv3, b2r, batch3):
    def body(q_ref, hs_ref, dp_ref, b_ref, bt_ref, g_ref, sums, cnts):
        i = pl.program_id(0)
        dinv = dp_ref[0, 0, :]
        qq = q_ref[...]
        out = (qq[0] + qq[1] + hs_ref[...]) * dinv[:, None] + b_ref[...]
        seg = bt_ref[0, 0, :]
        onehot = (
            lax.broadcasted_iota(jnp.int32, (G, BN), 0) == seg[None, :]
        ).astype(jnp.float32)
        s = jnp.dot(onehot, out, preferred_element_type=jnp.float32)
        c = jnp.sum(onehot, axis=1)[:, None]

        @pl.when(i == 0)
        def _():
            sums[...] = jnp.zeros_like(sums)
            cnts[...] = jnp.zeros_like(cnts)

        sums[...] += s
        cnts[...] += jnp.broadcast_to(c, (G, D))

        @pl.when(i == NB - 1)
        def _():
            g_ref[...] = sums[...] / jnp.maximum(cnts[...], 1.0)

    return pl.pallas_call(
        body,
        grid=(NB,),
        in_specs=[
            pl.BlockSpec((NC, BN, D), lambda i: (0, i, 0)),
            pl.BlockSpec((BN, D), lambda i: (i, 0)),
            pl.BlockSpec((1, 1, BN), lambda i: (i, 0, 0)),
            pl.BlockSpec((1, D), lambda i: (0, 0)),
            pl.BlockSpec((1, 1, BN), lambda i: (i, 0, 0)),
        ],
        out_specs=pl.BlockSpec((G, D), lambda i: (0, 0)),
        out_shape=jax.ShapeDtypeStruct((G, D), jnp.float32),
        scratch_shapes=[
            pltpu.VMEM((G, D), jnp.float32),
            pltpu.VMEM((G, D), jnp.float32),
        ],
    )(q, hs2, dinv3, b2r, batch3)


def kernel(x, edge_index, batch, W1, b1, W2, b2):
    src = edge_index[0]
    dst = edge_index[1]
    ar = jnp.arange(PAD, dtype=jnp.int32)
    srcp = jnp.concatenate([src, ar % N]).reshape(NW, KCH, CHUNK)
    dstp = jnp.concatenate([dst, N + ar % (NACC - N)]).reshape(NW, KCH, CHUNK)

    dp = _sc_degree(dstp).reshape(NC, NACC, D)
    hs1, dinv3 = _tc_in(x, W1, dp)
    p1 = _sc_edge_pass(hs1, srcp, dstp).reshape(NC, NACC, D)
    hs2 = _tc_mid(p1, hs1, dinv3, W2, b1.reshape(1, D))
    p2 = _sc_edge_pass(hs2, srcp, dstp).reshape(NC, NACC, D)
    return _tc_final(p2, hs2, dinv3, b2.reshape(1, D), batch.reshape(NB, 1, BN))


# speedup vs baseline: 28.0154x; 1.0042x over previous
import functools

import jax
import jax.numpy as jnp
from jax import lax
from jax.experimental import pallas as pl
from jax.experimental.pallas import tpu as pltpu
from jax.experimental.pallas import tpu_sc as plsc

N = 10000
E = 320000
D = 128
G = 64

NC = 2
NS = 16
NW = NC * NS
CHUNK = 128
EPW = 10240
KCH = EPW // CHUNK
EP = NW * EPW
PAD = EP - E
NACC = 10240
ZB = NACC // NS
DRAIN = NACC // NS

NSEG = 2
KS = KCH // NSEG

BN = 2000
NB = N // BN


def _mesh():
    return plsc.VectorSubcoreMesh(core_axis_name="c", subcore_axis_name="s")


def _sc_degree(dstp):
    ones_pay = jnp.ones((NW * CHUNK, D), jnp.float32)
    zer = jnp.zeros((NC * NACC, D), jnp.float32)
    kf = 8

    @functools.partial(
        pl.kernel,
        out_type=jax.ShapeDtypeStruct((NC * NACC, D), jnp.float32),
        mesh=_mesh(),
        scratch_types=[
            pltpu.VMEM((KCH, CHUNK), jnp.int32),
            pltpu.VMEM((CHUNK, D), jnp.float32),
            pltpu.VMEM_SHARED((NACC, D), jnp.float32),
            pltpu.SemaphoreType.DMA,
        ],
    )
    def k(dst_hbm, ones_hbm, z_hbm, out_hbm, dstv, onesv, acc, sem):
        cid = lax.axis_index("c")
        sid = lax.axis_index("s")
        wid = sid * NC + cid
        pltpu.sync_copy(z_hbm.at[pl.ds(cid * NACC + sid * ZB, ZB)],
                        acc.at[pl.ds(sid * ZB, ZB)])
        pltpu.sync_copy(dst_hbm.at[wid], dstv)
        pltpu.sync_copy(ones_hbm.at[pl.ds(wid * CHUNK, CHUNK)], onesv)
        plsc.subcore_barrier()

        @pl.loop(0, KCH // kf)
        def _(b):
            @pl.loop(0, kf)
            def _(i):
                pltpu.async_copy(onesv, acc.at[dstv.at[b * kf + i]], add=True,
                                 sem=sem)

            @pl.loop(0, kf)
            def _(i):
                pltpu.make_async_copy(onesv, acc.at[dstv.at[0]], sem).wait()

        plsc.subcore_barrier()
        pltpu.sync_copy(
            acc.at[pl.ds(sid * DRAIN, DRAIN)],
            out_hbm.at[pl.ds(cid * NACC + sid * DRAIN, DRAIN)],
        )

    return k(dstp, ones_pay, zer)


def _sc_edge_pass(hs, srcp, dstp):
    zer = jnp.zeros((NC * NACC, D), jnp.float32)

    @functools.partial(
        pl.kernel,
        out_type=jax.ShapeDtypeStruct((NC * NACC, D), jnp.float32),
        mesh=_mesh(),
        scratch_types=[
            pltpu.VMEM((KS, CHUNK), jnp.int32),
            pltpu.VMEM((KS, CHUNK), jnp.int32),
            pltpu.VMEM((CHUNK, D), jnp.float32),
            pltpu.VMEM((CHUNK, D), jnp.float32),
            pltpu.VMEM_SHARED((NACC, D), jnp.float32),
            pltpu.SemaphoreType.DMA,
            pltpu.SemaphoreType.DMA,
        ],
    )
    def k(hs_hbm, src_hbm, dst_hbm, z_hbm, out_hbm,
          srcv, dstv, buf0, buf1, acc, sem0, sem1):
        cid = lax.axis_index("c")
        sid = lax.axis_index("s")
        wid = sid * NC + cid
        pltpu.sync_copy(z_hbm.at[pl.ds(cid * NACC + sid * ZB, ZB)],
                        acc.at[pl.ds(sid * ZB, ZB)])
        plsc.subcore_barrier()

        @pl.loop(0, NSEG)
        def _(sg):
            pltpu.sync_copy(src_hbm.at[wid].at[pl.ds(sg * KS, KS)], srcv)
            pltpu.sync_copy(dst_hbm.at[wid].at[pl.ds(sg * KS, KS)], dstv)
            pltpu.async_copy(hs_hbm.at[srcv.at[0]], buf0, sem0)

            @pl.loop(0, KS // 2)
            def _(t):
                j0 = t * 2
                j1 = j0 + 1
                pltpu.async_copy(hs_hbm.at[srcv.at[j1]], buf1, sem1)
                pltpu.make_async_copy(hs_hbm.at[srcv.at[j0]], buf0, sem0).wait()
                pltpu.sync_copy(buf0, acc.at[dstv.at[j0]], add=True)

                @pl.when(j0 + 2 < KS)
                def _():
                    pltpu.async_copy(hs_hbm.at[srcv.at[j0 + 2]], buf0, sem0)

                pltpu.make_async_copy(hs_hbm.at[srcv.at[j1]], buf1, sem1).wait()
                pltpu.sync_copy(buf1, acc.at[dstv.at[j1]], add=True)

        plsc.subcore_barrier()
        pltpu.sync_copy(
            acc.at[pl.ds(sid * DRAIN, DRAIN)],
            out_hbm.at[pl.ds(cid * NACC + sid * DRAIN, DRAIN)],
        )

    return k(hs, srcp, dstp, zer)


def _deg_inv(dp_blk):
    deg = jnp.sum(dp_blk[0] + dp_blk[1], axis=1) * (1.0 / D) + 1.0
    return 1.0 / jnp.sqrt(deg)


def _tc_in(x, W1, dp):
    def body(x_ref, w_ref, p_ref, o_ref, d_ref):
        dinv = _deg_inv(p_ref[...])
        h = jnp.dot(x_ref[...], w_ref[...], preferred_element_type=jnp.float32)
        o_ref[...] = h * dinv[:, None]
        d_ref[0, 0, :] = dinv

    return pl.pallas_call(
        body,
        grid=(NB,),
        in_specs=[
            pl.BlockSpec((BN, D), lambda i: (i, 0)),
            pl.BlockSpec((D, D), lambda i: (0, 0)),
            pl.BlockSpec((NC, BN, D), lambda i: (0, i, 0)),
        ],
        out_specs=[
            pl.BlockSpec((BN, D), lambda i: (i, 0)),
            pl.BlockSpec((1, 1, BN), lambda i: (i, 0, 0)),
        ],
        out_shape=[
            jax.ShapeDtypeStruct((N, D), jnp.float32),
            jax.ShapeDtypeStruct((NB, 1, BN), jnp.float32),
        ],
    )(x, W1, dp)


def _tc_mid(p, hs1, dinv3, W2, b1r):
    def body(p_ref, hs_ref, dp_ref, w_ref, b_ref, o_ref):
        dinv = dp_ref[0, 0, :]
        pp = p_ref[...]
        agg = pp[0] + pp[1] + hs_ref[...]
        x2 = jnp.maximum(agg * dinv[:, None] + b_ref[...], 0.0)
        h2 = jnp.dot(x2, w_ref[...], preferred_element_type=jnp.float32)
        o_ref[...] = h2 * dinv[:, None]

    return pl.pallas_call(
        body,
        grid=(NB,),
        in_specs=[
            pl.BlockSpec((NC, BN, D), lambda i: (0, i, 0)),
            pl.BlockSpec((BN, D), lambda i: (i, 0)),
            pl.BlockSpec((1, 1, BN), lambda i: (i, 0, 0)),
            pl.BlockSpec((D, D), lambda i: (0, 0)),
            pl.BlockSpec((1, D), lambda i: (0, 0)),
        ],
        out_specs=pl.BlockSpec((BN, D), lambda i: (i, 0)),
        out_shape=jax.ShapeDtypeStruct((N, D), jnp.float32),
    )(p, hs1, dinv3, W2, b1r)


def _tc_final(q, hs2, dinv3, b2r, batch3):
    def body(q_ref, hs_ref, dp_ref, b_ref, bt_ref, g_ref, sums, cnts):
        i = pl.program_id(0)
        dinv = dp_ref[0, 0, :]
        qq = q_ref[...]
        out = (qq[0] + qq[1] + hs_ref[...]) * dinv[:, None] + b_ref[...]
        seg = bt_ref[0, 0, :]
        onehot = (
            lax.broadcasted_iota(jnp.int32, (G, BN), 0) == seg[None, :]
        ).astype(jnp.float32)
        s = jnp.dot(onehot, out, preferred_element_type=jnp.float32)
        c = jnp.sum(onehot, axis=1)[:, None]

        @pl.when(i == 0)
        def _():
            sums[...] = jnp.zeros_like(sums)
            cnts[...] = jnp.zeros_like(cnts)

        sums[...] += s
        cnts[...] += jnp.broadcast_to(c, (G, D))

        @pl.when(i == NB - 1)
        def _():
            g_ref[...] = sums[...] / jnp.maximum(cnts[...], 1.0)

    return pl.pallas_call(
        body,
        grid=(NB,),
        in_specs=[
            pl.BlockSpec((NC, BN, D), lambda i: (0, i, 0)),
            pl.BlockSpec((BN, D), lambda i: (i, 0)),
            pl.BlockSpec((1, 1, BN), lambda i: (i, 0, 0)),
            pl.BlockSpec((1, D), lambda i: (0, 0)),
            pl.BlockSpec((1, 1, BN), lambda i: (i, 0, 0)),
        ],
        out_specs=pl.BlockSpec((G, D), lambda i: (0, 0)),
        out_shape=jax.ShapeDtypeStruct((G, D), jnp.float32),
        scratch_shapes=[
            pltpu.VMEM((G, D), jnp.float32),
            pltpu.VMEM((G, D), jnp.float32),
        ],
    )(q, hs2, dinv3, b2r, batch3)


def kernel(x, edge_index, batch, W1, b1, W2, b2):
    src = edge_index[0]
    dst = edge_index[1]
    ar = jnp.arange(PAD, dtype=jnp.int32)
    srcp = jnp.concatenate([src, ar % N]).reshape(NW, KCH, CHUNK)
    dstp = jnp.concatenate([dst, N + ar % (NACC - N)]).reshape(NW, KCH, CHUNK)

    dp = _sc_degree(dstp).reshape(NC, NACC, D)
    hs1, dinv3 = _tc_in(x, W1, dp)
    p1 = _sc_edge_pass(hs1, srcp, dstp).reshape(NC, NACC, D)
    hs2 = _tc_mid(p1, hs1, dinv3, W2, b1.reshape(1, D))
    p2 = _sc_edge_pass(hs2, srcp, dstp).reshape(NC, NACC, D)
    return _tc_final(p2, hs2, dinv3, b2.reshape(1, D), batch.reshape(NB, 1, BN))
